# baseline (speedup 1.0000x reference)
import functools

import jax
import jax.numpy as jnp
from jax import lax
from jax.experimental import pallas as pl
from jax.experimental.pallas import tpu as pltpu
from jax.experimental.pallas import tpu_sc as plsc

NC = 2
NS = 16
NW = NC * NS

D = 128
DEG_W = 128


@functools.lru_cache(maxsize=None)
def _geom(n, e, ch=128, kb=40):
    nch = (((e + NW * ch - 1) // (NW * ch)) + kb - 1) // kb * kb
    nb = nch // kb
    epad = NW * nch * ch
    npad = ((n + NS * 8 - 1) // (NS * 8)) * (NS * 8)
    if npad == n and epad > e:
        npad += NS * 8
    rps = npad // NS
    return epad, ch, nch, kb, nb, npad, rps


_MESH = plsc.VectorSubcoreMesh(core_axis_name="c", subcore_axis_name="s")


def _fill(buf, rows, val, dw=D):
    @pl.loop(0, rows)
    def _(i):
        @pl.loop(0, dw // 16)
        def _(q):
            buf[i, pl.ds(q * 16, 16)] = jnp.full((16,), val, jnp.float32)


def _zero_slice(buf, ch, sh, base, rps):
    for t in range(rps // ch):
        pltpu.sync_copy(buf, sh.at[pl.ds(base + t * ch, ch)])
    rem = rps % ch
    if rem:
        pltpu.sync_copy(buf.at[pl.ds(0, rem)],
                        sh.at[pl.ds(base + (rps // ch) * ch, rem)])


def _agg_phase(feats, src, dst, wid, src_v, dst_v, rows_a, rows_b, acc_sh,
               gs_a, gs_b, ss_a, ss_b, nb, kb):
    @pl.loop(0, nb)
    def _(k):
        pltpu.sync_copy(src.at[wid * nb + k], src_v)
        pltpu.sync_copy(dst.at[wid * nb + k], dst_v)
        pltpu.async_copy(feats.at[src_v.at[0]], rows_a, gs_a)

        @pl.loop(0, kb // 2)
        def _(g):
            j0 = 2 * g
            j1 = 2 * g + 1

            @pl.when(g > 0)
            def _():
                pltpu.make_async_copy(
                    rows_b, acc_sh.at[dst_v.at[j0 - 1]], ss_b).wait()

            pltpu.async_copy(feats.at[src_v.at[j1]], rows_b, gs_b)
            pltpu.make_async_copy(
                feats.at[src_v.at[j0]], rows_a, gs_a).wait()
            pltpu.async_copy(
                rows_a, acc_sh.at[dst_v.at[j0]], ss_a, add=True)
            pltpu.make_async_copy(
                feats.at[src_v.at[j1]], rows_b, gs_b).wait()
            pltpu.make_async_copy(
                rows_a, acc_sh.at[dst_v.at[j0]], ss_a).wait()

            @pl.when(g < kb // 2 - 1)
            def _():
                pltpu.async_copy(feats.at[src_v.at[j0 + 2]], rows_a, gs_a)

            pltpu.async_copy(
                rows_b, acc_sh.at[dst_v.at[j1]], ss_b, add=True)

        pltpu.make_async_copy(
            rows_b, acc_sh.at[dst_v.at[kb - 1]], ss_b).wait()


@functools.lru_cache(maxsize=None)
def _make_sc_agg(n, e, with_deg):
    epad, ch, nch, kb, nb, npad, rps = _geom(n, e)

    out_type = [jax.ShapeDtypeStruct((NC, npad, D), jnp.float32)]
    if with_deg:
        out_type.append(jax.ShapeDtypeStruct((NC, npad, D), jnp.float32))
    scratch = [
        pltpu.VMEM((kb, ch), jnp.int32),
        pltpu.VMEM((kb, ch), jnp.int32),
        pltpu.VMEM((ch, D), jnp.float32),
        pltpu.VMEM((ch, D), jnp.float32),
        pltpu.VMEM_SHARED((npad, D), jnp.float32),
        pltpu.SemaphoreType.DMA,
        pltpu.SemaphoreType.DMA,
        pltpu.SemaphoreType.DMA,
        pltpu.SemaphoreType.DMA,
    ]

    def body(feats, src, dst, *rest):
        if with_deg:
            (acc_out, deg_out,
             src_v, dst_v, rows_a, rows_b, acc_sh, gs_a, gs_b, ss_a, ss_b) = rest
        else:
            (acc_out,
             src_v, dst_v, rows_a, rows_b, acc_sh, gs_a, gs_b, ss_a, ss_b) = rest
        core = lax.axis_index("c")
        sub = lax.axis_index("s")
        wid = sub * NC + core
        sl = pl.ds(sub * rps, rps)

        _fill(rows_a, ch, 0.0)
        _zero_slice(rows_a, ch, acc_sh, sub * rps, rps)
        plsc.subcore_barrier()

        _agg_phase(feats, src, dst, wid, src_v, dst_v, rows_a, rows_b,
                   acc_sh, gs_a, gs_b, ss_a, ss_b, nb, kb)

        plsc.subcore_barrier()
        pltpu.sync_copy(acc_sh.at[sl], acc_out.at[core, sl])

        if with_deg:
            _fill(rows_a, ch, 0.0)
            _zero_slice(rows_a, ch, acc_sh, sub * rps, rps)
            _fill(rows_a, ch, 1.0)
            plsc.subcore_barrier()

            @pl.loop(0, nb)
            def _(k):
                pltpu.sync_copy(dst.at[wid * nb + k], dst_v)

                @pl.loop(0, kb)
                def _(j):
                    pltpu.async_copy(rows_a, acc_sh.at[dst_v.at[j]], gs_a,
                                     add=True)

                @pl.loop(0, kb)
                def _(j):
                    pltpu.make_async_copy(rows_a, acc_sh.at[dst_v.at[j]],
                                          gs_a).wait()

            plsc.subcore_barrier()
            pltpu.sync_copy(acc_sh.at[sl], deg_out.at[core, sl])

    kern = pl.kernel(body, out_type=out_type, mesh=_MESH,
                     scratch_types=scratch)

    def run(feats, src_r, dst_r):
        return kern(feats, src_r, dst_r)

    return run


def _dot(a, w):
    return jax.lax.dot_general(
        a, w, (((1,), (0,)), ((), ())),
        preferred_element_type=jnp.float32)


def _mm_body(x_ref, w1_ref, w2_ref, o1_ref, o2_ref):
    x = x_ref[...]
    o1_ref[...] = _dot(x, w1_ref[...])
    o2_ref[...] = _dot(x, w2_ref[...])


def _make_mm2(n, bn):
    grid = (n // bn,)
    return pl.pallas_call(
        _mm_body,
        grid=grid,
        in_specs=[
            pl.BlockSpec((bn, D), lambda i: (i, 0)),
            pl.BlockSpec((D, D), lambda i: (0, 0)),
            pl.BlockSpec((D, D), lambda i: (0, 0)),
        ],
        out_specs=[
            pl.BlockSpec((bn, D), lambda i: (i, 0)),
            pl.BlockSpec((bn, D), lambda i: (i, 0)),
        ],
        out_shape=[
            jax.ShapeDtypeStruct((n, D), jnp.float32),
            jax.ShapeDtypeStruct((n, D), jnp.float32),
        ],
    )


def _mean(accp_ref, degp_ref):
    deg = degp_ref[0] + degp_ref[1]
    rdeg = 1.0 / jnp.maximum(deg[:, :1], 1.0)
    return (accp_ref[0] + accp_ref[1]) * rdeg


def _h_body(accp_ref, degp_ref, xr_ref, wl_ref, b_ref, o_ref, rdeg_ref):
    deg = degp_ref[0, :, :1] + degp_ref[1, :, :1]
    rdeg = 1.0 / jnp.maximum(deg, 1.0)
    mean = (accp_ref[0] + accp_ref[1]) * rdeg
    o_ref[...] = jax.nn.relu(_dot(mean, wl_ref[...]) + b_ref[...] + xr_ref[...])
    rdeg_ref[...] = jnp.broadcast_to(rdeg, rdeg_ref.shape)


def _make_h(n, bn):
    grid = (n // bn,)
    return pl.pallas_call(
        _h_body,
        grid=grid,
        in_specs=[
            pl.BlockSpec((NC, bn, D), lambda i: (0, i, 0)),
            pl.BlockSpec((NC, bn, DEG_W), lambda i: (0, i, 0)),
            pl.BlockSpec((bn, D), lambda i: (i, 0)),
            pl.BlockSpec((D, D), lambda i: (0, 0)),
            pl.BlockSpec((1, D), lambda i: (0, 0)),
        ],
        out_specs=[
            pl.BlockSpec((bn, D), lambda i: (i, 0)),
            pl.BlockSpec((bn, 8), lambda i: (i, 0)),
        ],
        out_shape=[
            jax.ShapeDtypeStruct((n, D), jnp.float32),
            jax.ShapeDtypeStruct((n, 8), jnp.float32),
        ],
    )


def _out_body(accp_ref, rdeg_ref, hr2_ref, hrd_ref, wl2_ref, wld_ref,
              b2_ref, bd_ref, oc_ref, od_ref):
    mean = (accp_ref[0] + accp_ref[1]) * rdeg_ref[:, :1]
    oc_ref[...] = _dot(mean, wl2_ref[...]) + b2_ref[...] + hr2_ref[...]
    od_ref[...] = _dot(mean, wld_ref[...]) + bd_ref[...] + hrd_ref[...]


def _make_out(n, bn):
    grid = (n // bn,)
    return pl.pallas_call(
        _out_body,
        grid=grid,
        in_specs=[
            pl.BlockSpec((NC, bn, D), lambda i: (0, i, 0)),
            pl.BlockSpec((bn, 8), lambda i: (i, 0)),
            pl.BlockSpec((bn, D), lambda i: (i, 0)),
            pl.BlockSpec((bn, D), lambda i: (i, 0)),
            pl.BlockSpec((D, D), lambda i: (0, 0)),
            pl.BlockSpec((D, D), lambda i: (0, 0)),
            pl.BlockSpec((1, D), lambda i: (0, 0)),
            pl.BlockSpec((1, D), lambda i: (0, 0)),
        ],
        out_specs=[
            pl.BlockSpec((bn, D), lambda i: (i, 0)),
            pl.BlockSpec((bn, D), lambda i: (i, 0)),
        ],
        out_shape=[
            jax.ShapeDtypeStruct((n, D), jnp.float32),
            jax.ShapeDtypeStruct((n, D), jnp.float32),
        ],
    )


def _pad_edges(src, dst, n, epad, e, npad):
    if epad > e:
        pad_i = jnp.arange(epad - e, dtype=jnp.int32)
        src = jnp.concatenate([src, pad_i % n])
        dst = jnp.concatenate([dst, n + pad_i % (npad - n)])
    return src, dst


def kernel(x, edge_index, Wl1, Wr1, b1, Wl2, Wr2, b2, Wld, Wrd, bd):
    n, d = x.shape
    e = edge_index.shape[1]
    src = edge_index[0]
    dst = edge_index[1]
    bn = 1000

    d_out = Wld.shape[0]
    pad = jnp.zeros((D, D - d_out), jnp.float32)
    wld_t = jnp.concatenate([Wld.T, pad], axis=1)
    wrd_t = jnp.concatenate([Wrd.T, pad], axis=1)
    bd_p = jnp.concatenate([bd, jnp.zeros((D - d_out,), jnp.float32)])

    agg = _make_sc_agg(n, e, False)
    agg_deg = _make_sc_agg(n, e, True)
    mm2 = _make_mm2(n, bn)
    mk_h = _make_h(n, bn)
    mk_out = _make_out(n, bn)

    epad, ch, nch, kb, nb, npad, rps = _geom(n, e)
    src, dst = _pad_edges(src, dst, n, epad, e, npad)
    src_r = src.reshape(NW * nb, kb, ch)
    dst_r = dst.reshape(NW * nb, kb, ch)

    a1p, degp = agg_deg(x, src_r, dst_r)
    xr, _ = mm2(x, Wr1.T, Wr1.T)
    h, rdeg8 = mk_h(a1p, degp, xr, Wl1.T, b1.reshape(1, D))

    (a2p,) = agg(h, src_r, dst_r)
    hr2, hrd = mm2(h, Wr2.T, wrd_t)
    y_class, y_domain_f = mk_out(a2p, rdeg8, hr2, hrd, Wl2.T, wld_t,
                                 b2.reshape(1, D), bd_p.reshape(1, D))
    return (y_class, y_domain_f[:, :d_out])


# speedup vs baseline: 1.0106x; 1.0106x over previous
import functools

import jax
import jax.numpy as jnp
from jax import lax
from jax.experimental import pallas as pl
from jax.experimental.pallas import tpu as pltpu
from jax.experimental.pallas import tpu_sc as plsc

NC = 2
NS = 16
NW = NC * NS

D = 128
DEG_W = 128


@functools.lru_cache(maxsize=None)
def _geom(n, e, ch=128, kb=40):
    nch = (((e + NW * ch - 1) // (NW * ch)) + kb - 1) // kb * kb
    nb = nch // kb
    epad = NW * nch * ch
    npad = ((n + NS * 8 - 1) // (NS * 8)) * (NS * 8)
    if npad == n and epad > e:
        npad += NS * 8
    rps = npad // NS
    return epad, ch, nch, kb, nb, npad, rps


_MESH = plsc.VectorSubcoreMesh(core_axis_name="c", subcore_axis_name="s")


def _fill(buf, rows, val, dw=D):
    @pl.loop(0, rows)
    def _(i):
        @pl.loop(0, dw // 16)
        def _(q):
            buf[i, pl.ds(q * 16, 16)] = jnp.full((16,), val, jnp.float32)


def _zero_slice(buf, ch, sh, base, rps):
    for t in range(rps // ch):
        pltpu.sync_copy(buf, sh.at[pl.ds(base + t * ch, ch)])
    rem = rps % ch
    if rem:
        pltpu.sync_copy(buf.at[pl.ds(0, rem)],
                        sh.at[pl.ds(base + (rps // ch) * ch, rem)])


def _agg_phase(feats, src, dst, wid, src_v, dst_v, rows_a, rows_b, acc_sh,
               gs_a, gs_b, ss_a, ss_b, nb, kb):
    @pl.loop(0, nb)
    def _(k):
        pltpu.sync_copy(src.at[wid * nb + k], src_v)
        pltpu.sync_copy(dst.at[wid * nb + k], dst_v)
        pltpu.async_copy(feats.at[src_v.at[0]], rows_a, gs_a)

        @pl.loop(0, kb // 2)
        def _(g):
            j0 = 2 * g
            j1 = 2 * g + 1

            @pl.when(g > 0)
            def _():
                pltpu.make_async_copy(
                    rows_b, acc_sh.at[dst_v.at[j0 - 1]], ss_b).wait()

            pltpu.async_copy(feats.at[src_v.at[j1]], rows_b, gs_b)
            pltpu.make_async_copy(
                feats.at[src_v.at[j0]], rows_a, gs_a).wait()
            pltpu.async_copy(
                rows_a, acc_sh.at[dst_v.at[j0]], ss_a, add=True)
            pltpu.make_async_copy(
                feats.at[src_v.at[j1]], rows_b, gs_b).wait()
            pltpu.make_async_copy(
                rows_a, acc_sh.at[dst_v.at[j0]], ss_a).wait()

            @pl.when(g < kb // 2 - 1)
            def _():
                pltpu.async_copy(feats.at[src_v.at[j0 + 2]], rows_a, gs_a)

            pltpu.async_copy(
                rows_b, acc_sh.at[dst_v.at[j1]], ss_b, add=True)

        pltpu.make_async_copy(
            rows_b, acc_sh.at[dst_v.at[kb - 1]], ss_b).wait()


@functools.lru_cache(maxsize=None)
def _make_sc_agg(n, e, with_deg):
    epad, ch, nch, kb, nb, npad, rps = _geom(n, e)

    out_type = [jax.ShapeDtypeStruct((NC, npad, D), jnp.float32)]
    if with_deg:
        out_type.append(jax.ShapeDtypeStruct((NC, npad, D), jnp.float32))
    scratch = [
        pltpu.VMEM((kb, ch), jnp.int32),
        pltpu.VMEM((kb, ch), jnp.int32),
        pltpu.VMEM((ch, D), jnp.float32),
        pltpu.VMEM((ch, D), jnp.float32),
        pltpu.VMEM_SHARED((npad, D), jnp.float32),
        pltpu.SemaphoreType.DMA,
        pltpu.SemaphoreType.DMA,
        pltpu.SemaphoreType.DMA,
        pltpu.SemaphoreType.DMA,
    ]

    def body(feats, src, dst, *rest):
        if with_deg:
            (acc_out, deg_out,
             src_v, dst_v, rows_a, rows_b, acc_sh, gs_a, gs_b, ss_a, ss_b) = rest
        else:
            (acc_out,
             src_v, dst_v, rows_a, rows_b, acc_sh, gs_a, gs_b, ss_a, ss_b) = rest
        core = lax.axis_index("c")
        sub = lax.axis_index("s")
        wid = sub * NC + core
        sl = pl.ds(sub * rps, rps)

        _fill(rows_a, ch, 0.0)
        _zero_slice(rows_a, ch, acc_sh, sub * rps, rps)
        plsc.subcore_barrier()

        _agg_phase(feats, src, dst, wid, src_v, dst_v, rows_a, rows_b,
                   acc_sh, gs_a, gs_b, ss_a, ss_b, nb, kb)

        plsc.subcore_barrier()
        pltpu.sync_copy(acc_sh.at[sl], acc_out.at[core, sl])

        if with_deg:
            _fill(rows_a, ch, 0.0)
            _zero_slice(rows_a, ch, acc_sh, sub * rps, rps)
            _fill(rows_a, ch, 1.0)
            plsc.subcore_barrier()

            @pl.loop(0, nb)
            def _(k):
                pltpu.sync_copy(dst.at[wid * nb + k], dst_v)

                @pl.loop(0, kb)
                def _(j):
                    pltpu.async_copy(rows_a, acc_sh.at[dst_v.at[j]], gs_a,
                                     add=True)

                @pl.loop(0, kb)
                def _(j):
                    pltpu.make_async_copy(rows_a, acc_sh.at[dst_v.at[j]],
                                          gs_a).wait()

            plsc.subcore_barrier()
            pltpu.sync_copy(acc_sh.at[sl], deg_out.at[core, sl])

    kern = pl.kernel(body, out_type=out_type, mesh=_MESH,
                     scratch_types=scratch)

    def run(feats, src_r, dst_r):
        return kern(feats, src_r, dst_r)

    return run


def _dot(a, w):
    return jax.lax.dot_general(
        a, w, (((1,), (0,)), ((), ())),
        preferred_element_type=jnp.float32)


def _mm_body(x_ref, w1_ref, w2_ref, o1_ref, o2_ref):
    x = x_ref[...]
    o1_ref[...] = _dot(x, w1_ref[...])
    o2_ref[...] = _dot(x, w2_ref[...])


def _make_mm2(n, bn):
    grid = (n // bn,)
    return pl.pallas_call(
        _mm_body,
        grid=grid,
        in_specs=[
            pl.BlockSpec((bn, D), lambda i: (i, 0)),
            pl.BlockSpec((D, D), lambda i: (0, 0)),
            pl.BlockSpec((D, D), lambda i: (0, 0)),
        ],
        out_specs=[
            pl.BlockSpec((bn, D), lambda i: (i, 0)),
            pl.BlockSpec((bn, D), lambda i: (i, 0)),
        ],
        out_shape=[
            jax.ShapeDtypeStruct((n, D), jnp.float32),
            jax.ShapeDtypeStruct((n, D), jnp.float32),
        ],
    )


def _mean(accp_ref, degp_ref):
    deg = degp_ref[0] + degp_ref[1]
    rdeg = 1.0 / jnp.maximum(deg[:, :1], 1.0)
    return (accp_ref[0] + accp_ref[1]) * rdeg


def _h_body(accp_ref, degp_ref, xr_ref, wl_ref, b_ref, o_ref, rdeg_ref):
    deg = degp_ref[0, :, :1] + degp_ref[1, :, :1]
    rdeg = 1.0 / jnp.maximum(deg, 1.0)
    mean = (accp_ref[0] + accp_ref[1]) * rdeg
    o_ref[...] = jax.nn.relu(_dot(mean, wl_ref[...]) + b_ref[...] + xr_ref[...])
    rdeg_ref[...] = jnp.broadcast_to(rdeg, rdeg_ref.shape)


def _make_h(n, bn):
    grid = (n // bn,)
    return pl.pallas_call(
        _h_body,
        grid=grid,
        in_specs=[
            pl.BlockSpec((NC, bn, D), lambda i: (0, i, 0)),
            pl.BlockSpec((NC, bn, DEG_W), lambda i: (0, i, 0)),
            pl.BlockSpec((bn, D), lambda i: (i, 0)),
            pl.BlockSpec((D, D), lambda i: (0, 0)),
            pl.BlockSpec((1, D), lambda i: (0, 0)),
        ],
        out_specs=[
            pl.BlockSpec((bn, D), lambda i: (i, 0)),
            pl.BlockSpec((bn, 8), lambda i: (i, 0)),
        ],
        out_shape=[
            jax.ShapeDtypeStruct((n, D), jnp.float32),
            jax.ShapeDtypeStruct((n, 8), jnp.float32),
        ],
    )


def _out_body(accp_ref, rdeg_ref, hr2_ref, hrd_ref, wl2_ref, wld_ref,
              b2_ref, bd_ref, oc_ref, od_ref):
    mean = (accp_ref[0] + accp_ref[1]) * rdeg_ref[:, :1]
    oc_ref[...] = _dot(mean, wl2_ref[...]) + b2_ref[...] + hr2_ref[...]
    od_ref[...] = _dot(mean, wld_ref[...]) + bd_ref[...] + hrd_ref[...]


def _make_out(n, bn):
    grid = (n // bn,)
    return pl.pallas_call(
        _out_body,
        grid=grid,
        in_specs=[
            pl.BlockSpec((NC, bn, D), lambda i: (0, i, 0)),
            pl.BlockSpec((bn, 8), lambda i: (i, 0)),
            pl.BlockSpec((bn, D), lambda i: (i, 0)),
            pl.BlockSpec((bn, D), lambda i: (i, 0)),
            pl.BlockSpec((D, D), lambda i: (0, 0)),
            pl.BlockSpec((D, D), lambda i: (0, 0)),
            pl.BlockSpec((1, D), lambda i: (0, 0)),
            pl.BlockSpec((1, D), lambda i: (0, 0)),
        ],
        out_specs=[
            pl.BlockSpec((bn, D), lambda i: (i, 0)),
            pl.BlockSpec((bn, D), lambda i: (i, 0)),
        ],
        out_shape=[
            jax.ShapeDtypeStruct((n, D), jnp.float32),
            jax.ShapeDtypeStruct((n, D), jnp.float32),
        ],
    )


def _pad_edges(src, dst, n, epad, e, npad):
    if epad > e:
        pad_i = jnp.arange(epad - e, dtype=jnp.int32)
        src = jnp.concatenate([src, pad_i % n])
        dst = jnp.concatenate([dst, n + pad_i % (npad - n)])
    return src, dst


def kernel(x, edge_index, Wl1, Wr1, b1, Wl2, Wr2, b2, Wld, Wrd, bd):
    n, d = x.shape
    e = edge_index.shape[1]
    src = edge_index[0]
    dst = edge_index[1]
    bn = 2000

    d_out = Wld.shape[0]
    pad = jnp.zeros((D, D - d_out), jnp.float32)
    wld_t = jnp.concatenate([Wld.T, pad], axis=1)
    wrd_t = jnp.concatenate([Wrd.T, pad], axis=1)
    bd_p = jnp.concatenate([bd, jnp.zeros((D - d_out,), jnp.float32)])

    agg = _make_sc_agg(n, e, False)
    agg_deg = _make_sc_agg(n, e, True)
    mm2 = _make_mm2(n, bn)
    mk_h = _make_h(n, bn)
    mk_out = _make_out(n, bn)

    epad, ch, nch, kb, nb, npad, rps = _geom(n, e)
    src, dst = _pad_edges(src, dst, n, epad, e, npad)
    src_r = src.reshape(NW * nb, kb, ch)
    dst_r = dst.reshape(NW * nb, kb, ch)

    a1p, degp = agg_deg(x, src_r, dst_r)
    xr, _ = mm2(x, Wr1.T, Wr1.T)
    h, rdeg8 = mk_h(a1p, degp, xr, Wl1.T, b1.reshape(1, D))

    (a2p,) = agg(h, src_r, dst_r)
    hr2, hrd = mm2(h, Wr2.T, wrd_t)
    y_class, y_domain_f = mk_out(a2p, rdeg8, hr2, hrd, Wl2.T, wld_t,
                                 b2.reshape(1, D), bd_p.reshape(1, D))
    return (y_class, y_domain_f[:, :d_out])


# speedup vs baseline: 1.0122x; 1.0015x over previous
import functools

import jax
import jax.numpy as jnp
from jax import lax
from jax.experimental import pallas as pl
from jax.experimental.pallas import tpu as pltpu
from jax.experimental.pallas import tpu_sc as plsc

NC = 2
NS = 16
NW = NC * NS

D = 128
DEG_W = 128


@functools.lru_cache(maxsize=None)
def _geom(n, e, ch=128, kb=40):
    nch = (((e + NW * ch - 1) // (NW * ch)) + kb - 1) // kb * kb
    nb = nch // kb
    epad = NW * nch * ch
    npad = ((n + NS * 8 - 1) // (NS * 8)) * (NS * 8)
    if npad == n and epad > e:
        npad += NS * 8
    rps = npad // NS
    return epad, ch, nch, kb, nb, npad, rps


_MESH = plsc.VectorSubcoreMesh(core_axis_name="c", subcore_axis_name="s")


def _fill(buf, rows, val, dw=D):
    @pl.loop(0, rows)
    def _(i):
        @pl.loop(0, dw // 16)
        def _(q):
            buf[i, pl.ds(q * 16, 16)] = jnp.full((16,), val, jnp.float32)


def _zero_slice(buf, ch, sh, base, rps):
    for t in range(rps // ch):
        pltpu.sync_copy(buf, sh.at[pl.ds(base + t * ch, ch)])
    rem = rps % ch
    if rem:
        pltpu.sync_copy(buf.at[pl.ds(0, rem)],
                        sh.at[pl.ds(base + (rps // ch) * ch, rem)])


def _agg_phase(feats, src, dst, wid, src_v, dst_v, rows_a, rows_b, acc_sh,
               gs_a, gs_b, ss_a, ss_b, nb, kb):
    @pl.loop(0, nb)
    def _(k):
        pltpu.sync_copy(src.at[wid * nb + k], src_v)
        pltpu.sync_copy(dst.at[wid * nb + k], dst_v)
        pltpu.async_copy(feats.at[src_v.at[0]], rows_a, gs_a)

        @pl.loop(0, kb // 2)
        def _(g):
            j0 = 2 * g
            j1 = 2 * g + 1

            @pl.when(g > 0)
            def _():
                pltpu.make_async_copy(
                    rows_b, acc_sh.at[dst_v.at[j0 - 1]], ss_b).wait()

            pltpu.async_copy(feats.at[src_v.at[j1]], rows_b, gs_b)
            pltpu.make_async_copy(
                feats.at[src_v.at[j0]], rows_a, gs_a).wait()
            pltpu.async_copy(
                rows_a, acc_sh.at[dst_v.at[j0]], ss_a, add=True)
            pltpu.make_async_copy(
                feats.at[src_v.at[j1]], rows_b, gs_b).wait()
            pltpu.make_async_copy(
                rows_a, acc_sh.at[dst_v.at[j0]], ss_a).wait()

            @pl.when(g < kb // 2 - 1)
            def _():
                pltpu.async_copy(feats.at[src_v.at[j0 + 2]], rows_a, gs_a)

            pltpu.async_copy(
                rows_b, acc_sh.at[dst_v.at[j1]], ss_b, add=True)

        pltpu.make_async_copy(
            rows_b, acc_sh.at[dst_v.at[kb - 1]], ss_b).wait()


@functools.lru_cache(maxsize=None)
def _make_sc_agg(n, e, with_deg):
    epad, ch, nch, kb, nb, npad, rps = _geom(n, e)

    out_type = [jax.ShapeDtypeStruct((NC, npad, D), jnp.float32)]
    if with_deg:
        out_type.append(jax.ShapeDtypeStruct((NC, npad, D), jnp.float32))
    scratch = [
        pltpu.VMEM((kb, ch), jnp.int32),
        pltpu.VMEM((kb, ch), jnp.int32),
        pltpu.VMEM((ch, D), jnp.float32),
        pltpu.VMEM((ch, D), jnp.float32),
        pltpu.VMEM_SHARED((npad, D), jnp.float32),
        pltpu.SemaphoreType.DMA,
        pltpu.SemaphoreType.DMA,
        pltpu.SemaphoreType.DMA,
        pltpu.SemaphoreType.DMA,
    ]

    def body(feats, src, dst, *rest):
        if with_deg:
            (acc_out, deg_out,
             src_v, dst_v, rows_a, rows_b, acc_sh, gs_a, gs_b, ss_a, ss_b) = rest
        else:
            (acc_out,
             src_v, dst_v, rows_a, rows_b, acc_sh, gs_a, gs_b, ss_a, ss_b) = rest
        core = lax.axis_index("c")
        sub = lax.axis_index("s")
        wid = sub * NC + core
        sl = pl.ds(sub * rps, rps)

        _fill(rows_a, ch, 0.0)
        _zero_slice(rows_a, ch, acc_sh, sub * rps, rps)
        plsc.subcore_barrier()

        _agg_phase(feats, src, dst, wid, src_v, dst_v, rows_a, rows_b,
                   acc_sh, gs_a, gs_b, ss_a, ss_b, nb, kb)

        plsc.subcore_barrier()
        pltpu.sync_copy(acc_sh.at[sl], acc_out.at[core, sl])

        if with_deg:
            _fill(rows_a, ch, 0.0)
            _zero_slice(rows_a, ch, acc_sh, sub * rps, rps)
            _fill(rows_a, ch, 1.0)
            plsc.subcore_barrier()

            @pl.loop(0, nb)
            def _(k):
                pltpu.sync_copy(dst.at[wid * nb + k], dst_v)

                @pl.loop(0, kb)
                def _(j):
                    pltpu.async_copy(rows_a, acc_sh.at[dst_v.at[j]], gs_a,
                                     add=True)

                @pl.loop(0, kb)
                def _(j):
                    pltpu.make_async_copy(rows_a, acc_sh.at[dst_v.at[j]],
                                          gs_a).wait()

            plsc.subcore_barrier()
            pltpu.sync_copy(acc_sh.at[sl], deg_out.at[core, sl])

    kern = pl.kernel(body, out_type=out_type, mesh=_MESH,
                     scratch_types=scratch)

    def run(feats, src_r, dst_r):
        return kern(feats, src_r, dst_r)

    return run


def _dot(a, w):
    return jax.lax.dot_general(
        a, w, (((1,), (0,)), ((), ())),
        preferred_element_type=jnp.float32)


def _mm_body(x_ref, w1_ref, w2_ref, o1_ref, o2_ref):
    x = x_ref[...]
    o1_ref[...] = _dot(x, w1_ref[...])
    o2_ref[...] = _dot(x, w2_ref[...])


def _make_mm2(n, bn):
    grid = (n // bn,)
    return pl.pallas_call(
        _mm_body,
        grid=grid,
        in_specs=[
            pl.BlockSpec((bn, D), lambda i: (i, 0)),
            pl.BlockSpec((D, D), lambda i: (0, 0)),
            pl.BlockSpec((D, D), lambda i: (0, 0)),
        ],
        out_specs=[
            pl.BlockSpec((bn, D), lambda i: (i, 0)),
            pl.BlockSpec((bn, D), lambda i: (i, 0)),
        ],
        out_shape=[
            jax.ShapeDtypeStruct((n, D), jnp.float32),
            jax.ShapeDtypeStruct((n, D), jnp.float32),
        ],
    )


def _mean(accp_ref, degp_ref):
    deg = degp_ref[0] + degp_ref[1]
    rdeg = 1.0 / jnp.maximum(deg[:, :1], 1.0)
    return (accp_ref[0] + accp_ref[1]) * rdeg


def _h_body(accp_ref, degp_ref, xr_ref, wl_ref, b_ref, o_ref, rdeg_ref):
    deg = degp_ref[0, :, :1] + degp_ref[1, :, :1]
    rdeg = 1.0 / jnp.maximum(deg, 1.0)
    mean = (accp_ref[0] + accp_ref[1]) * rdeg
    o_ref[...] = jax.nn.relu(_dot(mean, wl_ref[...]) + b_ref[...] + xr_ref[...])
    rdeg_ref[...] = jnp.broadcast_to(rdeg, rdeg_ref.shape)


def _make_h(n, bn):
    grid = (n // bn,)
    return pl.pallas_call(
        _h_body,
        grid=grid,
        in_specs=[
            pl.BlockSpec((NC, bn, D), lambda i: (0, i, 0)),
            pl.BlockSpec((NC, bn, DEG_W), lambda i: (0, i, 0)),
            pl.BlockSpec((bn, D), lambda i: (i, 0)),
            pl.BlockSpec((D, D), lambda i: (0, 0)),
            pl.BlockSpec((1, D), lambda i: (0, 0)),
        ],
        out_specs=[
            pl.BlockSpec((bn, D), lambda i: (i, 0)),
            pl.BlockSpec((bn, 8), lambda i: (i, 0)),
        ],
        out_shape=[
            jax.ShapeDtypeStruct((n, D), jnp.float32),
            jax.ShapeDtypeStruct((n, 8), jnp.float32),
        ],
    )


def _out_body(accp_ref, rdeg_ref, hr2_ref, hrd_ref, wl2_ref, wld_ref,
              b2_ref, bd_ref, oc_ref, od_ref):
    mean = (accp_ref[0] + accp_ref[1]) * rdeg_ref[:, :1]
    oc_ref[...] = _dot(mean, wl2_ref[...]) + b2_ref[...] + hr2_ref[...]
    od_ref[...] = _dot(mean, wld_ref[...]) + bd_ref[...] + hrd_ref[...]


def _make_out(n, bn):
    grid = (n // bn,)
    return pl.pallas_call(
        _out_body,
        grid=grid,
        in_specs=[
            pl.BlockSpec((NC, bn, D), lambda i: (0, i, 0)),
            pl.BlockSpec((bn, 8), lambda i: (i, 0)),
            pl.BlockSpec((bn, D), lambda i: (i, 0)),
            pl.BlockSpec((bn, D), lambda i: (i, 0)),
            pl.BlockSpec((D, D), lambda i: (0, 0)),
            pl.BlockSpec((D, D), lambda i: (0, 0)),
            pl.BlockSpec((1, D), lambda i: (0, 0)),
            pl.BlockSpec((1, D), lambda i: (0, 0)),
        ],
        out_specs=[
            pl.BlockSpec((bn, D), lambda i: (i, 0)),
            pl.BlockSpec((bn, D), lambda i: (i, 0)),
        ],
        out_shape=[
            jax.ShapeDtypeStruct((n, D), jnp.float32),
            jax.ShapeDtypeStruct((n, D), jnp.float32),
        ],
    )


def _pad_edges(src, dst, n, epad, e, npad):
    if epad > e:
        pad_i = jnp.arange(epad - e, dtype=jnp.int32)
        src = jnp.concatenate([src, pad_i % n])
        dst = jnp.concatenate([dst, n + pad_i % (npad - n)])
    return src, dst


def kernel(x, edge_index, Wl1, Wr1, b1, Wl2, Wr2, b2, Wld, Wrd, bd):
    n, d = x.shape
    e = edge_index.shape[1]
    src = edge_index[0]
    dst = edge_index[1]
    bn = 5000

    d_out = Wld.shape[0]
    pad = jnp.zeros((D, D - d_out), jnp.float32)
    wld_t = jnp.concatenate([Wld.T, pad], axis=1)
    wrd_t = jnp.concatenate([Wrd.T, pad], axis=1)
    bd_p = jnp.concatenate([bd, jnp.zeros((D - d_out,), jnp.float32)])

    agg = _make_sc_agg(n, e, False)
    agg_deg = _make_sc_agg(n, e, True)
    mm2 = _make_mm2(n, bn)
    mk_h = _make_h(n, bn)
    mk_out = _make_out(n, bn)

    epad, ch, nch, kb, nb, npad, rps = _geom(n, e)
    src, dst = _pad_edges(src, dst, n, epad, e, npad)
    src_r = src.reshape(NW * nb, kb, ch)
    dst_r = dst.reshape(NW * nb, kb, ch)

    a1p, degp = agg_deg(x, src_r, dst_r)
    xr, _ = mm2(x, Wr1.T, Wr1.T)
    h, rdeg8 = mk_h(a1p, degp, xr, Wl1.T, b1.reshape(1, D))

    (a2p,) = agg(h, src_r, dst_r)
    hr2, hrd = mm2(h, Wr2.T, wrd_t)
    y_class, y_domain_f = mk_out(a2p, rdeg8, hr2, hrd, Wl2.T, wld_t,
                                 b2.reshape(1, D), bd_p.reshape(1, D))
    return (y_class, y_domain_f[:, :d_out])
